# baseline (speedup 1.0000x reference)
import dataclasses
import functools

import jax
import jax.numpy as jnp
from jax import lax
from jax.experimental import pallas as pl
from jax.experimental.pallas import tpu as pltpu
from jax.experimental.pallas import tpu_sc as plsc

B, N1, N2 = 16, 1024, 4096
C1, C2 = 256, 128
F1, F2 = 256, 256
EPS = 1e-7

T2 = 256
NT2 = N2 // T2
SUBQ = 128
NSTEP = (B * N2) // SUBQ
CW = C1 // 2
RPT = N1 // NT2
LANES = 16
TM = 512

_HI = -65536


def _bf16_topbits(x):
    t = lax.bitcast_convert_type(x, jnp.int32)
    lsb = jnp.bitwise_and(lax.shift_right_logical(t, 16), 1)
    return jnp.bitwise_and(t + 0x7FFF + lsb, _HI)


def _three_nn_body(xyz1_ref, xyz2t_ref, p1_ref, idx_ref, w_ref, tab_ref):
    b = pl.program_id(0)
    pts = p1_ref[0]
    lo = _bf16_topbits(pts[:, :CW])
    hi = _bf16_topbits(pts[:, CW:])
    tab_ref[...] = jnp.bitwise_or(lax.shift_right_logical(lo, 16), hi)

    xyz1 = xyz1_ref[0]
    q = xyz2t_ref[0]
    inf = jnp.float32(jnp.inf)
    RC = 128
    iota_l = lax.broadcasted_iota(jnp.int32, (RC, T2), 0)
    best_v = [jnp.full((1, T2), inf, jnp.float32) for _ in range(3)]
    best_i = [jnp.full((1, T2), N1, jnp.int32) for _ in range(3)]
    for c in range(N1 // RC):
        xc = xyz1[c * RC:(c + 1) * RC]
        dx = xc[:, 0:1] - q[0:1, :]
        dy = xc[:, 1:2] - q[1:2, :]
        dz = xc[:, 2:3] - q[2:3, :]
        cur = (dx * dx + dy * dy) + dz * dz
        for _ in range(3):
            m = jnp.min(cur, axis=0, keepdims=True)
            cand = jnp.where(cur == m, iota_l, RC)
            il = jnp.min(cand, axis=0, keepdims=True)
            cur = jnp.where(iota_l == il, inf, cur)
            ig = il + c * RC
            for j in range(3):
                sw = (m < best_v[j]) | ((m == best_v[j]) & (ig < best_i[j]))
                nv = jnp.where(sw, m, best_v[j])
                ni = jnp.where(sw, ig, best_i[j])
                m = jnp.where(sw, best_v[j], m)
                ig = jnp.where(sw, best_i[j], ig)
                best_v[j], best_i[j] = nv, ni
    rcp = [1.0 / jnp.maximum(v, EPS) for v in best_v]
    norm = rcp[0] + rcp[1] + rcp[2]
    gidx = jnp.concatenate([i + b * N1 for i in best_i], axis=0)
    wval = jnp.concatenate([r / norm for r in rcp], axis=0)
    for h in range(T2 // SUBQ):
        sl = slice(h * SUBQ, (h + 1) * SUBQ)
        idx_ref[0, 0, h] = gidx[:, sl]
        w_ref[0, 0, h] = wval[:, sl]


def _three_nn(xyz1, xyz2t, points1, nb):
    return pl.pallas_call(
        _three_nn_body,
        grid=(nb, NT2),
        in_specs=[
            pl.BlockSpec((1, N1, 3), lambda b, t: (b, 0, 0)),
            pl.BlockSpec((1, 3, T2), lambda b, t: (b, 0, t)),
            pl.BlockSpec((1, RPT, C1), lambda b, t: (b, t, 0)),
        ],
        out_specs=[
            pl.BlockSpec((1, 1, T2 // SUBQ, 3, SUBQ), lambda b, t: (b, t, 0, 0, 0)),
            pl.BlockSpec((1, 1, T2 // SUBQ, 3, SUBQ), lambda b, t: (b, t, 0, 0, 0)),
            pl.BlockSpec((RPT, CW), lambda b, t: (b * NT2 + t, 0)),
        ],
        out_shape=[
            jax.ShapeDtypeStruct((nb, NT2, T2 // SUBQ, 3, SUBQ), jnp.int32),
            jax.ShapeDtypeStruct((nb, NT2, T2 // SUBQ, 3, SUBQ), jnp.float32),
            jax.ShapeDtypeStruct((nb * N1, CW), jnp.int32),
        ],
    )(xyz1, xyz2t, points1)


def _sc_interp(table, idx3, w3, nrow, nstep):
    mesh = plsc.VectorSubcoreMesh(core_axis_name="c", subcore_axis_name="s")
    cp = pltpu.CompilerParams()
    if "needs_layout_passes" in pltpu.CompilerParams.__dataclass_fields__:
        cp = dataclasses.replace(cp, needs_layout_passes=False)

    @functools.partial(
        pl.kernel,
        out_type=jax.ShapeDtypeStruct((nrow, CW), jnp.int32),
        mesh=mesh,
        compiler_params=cp,
        scratch_types=[
            pltpu.VMEM((SUBQ, CW), jnp.int32),
            pltpu.VMEM((SUBQ, CW), jnp.int32),
            pltpu.VMEM((SUBQ, CW), jnp.int32),
        ],
    )
    def run(table_hbm, idx_hbm, w_hbm, out_hbm, r0, r1, r2):
        rows = (r0, r1, r2)

        def body(idx_vm, w_vm, out_vm):
            for k in range(3):
                pltpu.sync_copy(table_hbm.at[idx_vm.at[0, k]], rows[k])

            @pl.loop(0, SUBQ)
            def _(qv):
                z16 = jnp.zeros((LANES,), jnp.int32)
                qi = jnp.full((LANES,), qv, jnp.int32)
                w = []
                for k in range(3):
                    wf = plsc.load_gather(
                        w_vm, [z16, jnp.full((LANES,), k, jnp.int32), qi]
                    )
                    w.append(
                        plsc.pack(wf, wf, format=plsc.PackFormat.INTERLEAVED)
                    )
                for c in range(CW // LANES):
                    cs = pl.ds(c * LANES, LANES)
                    acc = w[0] * plsc.bitcast(rows[0][qv, cs], jnp.bfloat16)
                    acc += w[1] * plsc.bitcast(rows[1][qv, cs], jnp.bfloat16)
                    acc += w[2] * plsc.bitcast(rows[2][qv, cs], jnp.bfloat16)
                    out_vm[qv, cs] = plsc.bitcast(acc, jnp.int32)

        pltpu.emit_pipeline(
            body,
            grid=(nstep,),
            in_specs=[
                pl.BlockSpec((1, 3, SUBQ), lambda i: (i, 0, 0)),
                pl.BlockSpec((1, 3, SUBQ), lambda i: (i, 0, 0)),
            ],
            out_specs=[pl.BlockSpec((SUBQ, CW), lambda i: (i, 0))],
            core_axis_name=("c", "s"),
            dimension_semantics=(pltpu.PARALLEL,),
        )(idx_hbm, w_hbm, out_hbm)

    return run(table, idx3, w3)


def _mlp_body(xp_ref, p_ref, w1a_ref, w1b_ref, b1_ref, w2_ref, b2_ref, o_ref):
    wd = xp_ref[...]
    x_lo = lax.bitcast_convert_type(lax.shift_left(wd, 16), jnp.float32)
    x_hi = lax.bitcast_convert_type(jnp.bitwise_and(wd, _HI), jnp.float32)
    h = jnp.dot(x_lo.astype(jnp.bfloat16), w1a_ref[:CW],
                preferred_element_type=jnp.float32)
    h += jnp.dot(x_hi.astype(jnp.bfloat16), w1a_ref[CW:],
                 preferred_element_type=jnp.float32)
    h += jnp.dot(p_ref[...].astype(jnp.bfloat16), w1b_ref[...],
                 preferred_element_type=jnp.float32)
    h = jnp.maximum(h + b1_ref[...], 0.0)
    o = jnp.dot(h, w2_ref[...], preferred_element_type=jnp.float32)
    o_ref[...] = jnp.maximum(o + b2_ref[...], 0.0)


def _mlp(xp, p, w1a, w1b, b1, w2, b2):
    nrow = xp.shape[0]
    return pl.pallas_call(
        _mlp_body,
        grid=(nrow // TM,),
        in_specs=[
            pl.BlockSpec((TM, CW), lambda i: (i, 0)),
            pl.BlockSpec((TM, C2), lambda i: (i, 0)),
            pl.BlockSpec((C1, F1), lambda i: (0, 0)),
            pl.BlockSpec((C2, F1), lambda i: (0, 0)),
            pl.BlockSpec((1, F1), lambda i: (0, 0)),
            pl.BlockSpec((F1, F2), lambda i: (0, 0)),
            pl.BlockSpec((1, F2), lambda i: (0, 0)),
        ],
        out_specs=pl.BlockSpec((TM, F2), lambda i: (i, 0)),
        out_shape=jax.ShapeDtypeStruct((nrow, F2), jnp.float32),
    )(xp, p, w1a, w1b, b1, w2, b2)


GROUPS = (10, 6)


def kernel(inputs_0, inputs_1, W1, b1, W2, b2):
    xyz1 = inputs_0[:, :, 0:3]
    points1 = inputs_0[:, :, 3:]
    xyz2 = inputs_1[:, :, 0:3]
    points2 = inputs_1[:, :, 3:]
    xyz2t = jnp.transpose(xyz2, (0, 2, 1))

    w1a = W1[:C1].astype(jnp.bfloat16)
    w1b = W1[C1:].astype(jnp.bfloat16)
    b1r, b2r = b1.reshape(1, F1), b2.reshape(1, F2)

    hs = []
    base = 0
    for bg in GROUPS:
        sl = slice(base, base + bg)
        base += bg
        idx3, w3, table = _three_nn(xyz1[sl], xyz2t[sl], points1[sl], bg)
        nstep = (bg * N2) // SUBQ
        interp = _sc_interp(
            table,
            idx3.reshape(nstep, 3, SUBQ),
            w3.reshape(nstep, 3, SUBQ),
            bg * N2,
            nstep,
        )
        hs.append(
            _mlp(interp, points2[sl].reshape(bg * N2, C2),
                 w1a, w1b, b1r, W2, b2r)
        )
    h = jnp.concatenate(hs, axis=0)
    return h.reshape(B, N2, F2), xyz2


# speedup vs baseline: 1.1188x; 1.1188x over previous
import dataclasses
import functools

import jax
import jax.numpy as jnp
from jax import lax
from jax.experimental import pallas as pl
from jax.experimental.pallas import tpu as pltpu
from jax.experimental.pallas import tpu_sc as plsc

B, N1, N2 = 16, 1024, 4096
C1, C2 = 256, 128
F1, F2 = 256, 256
EPS = 1e-7

T2 = 256
NT2 = N2 // T2
SUBQ = 128
NSTEP = (B * N2) // SUBQ
CW = C1 // 2
RPT = N1 // NT2
LANES = 16
TM = 512

_HI = -65536


def _bf16_topbits(x):
    t = lax.bitcast_convert_type(x, jnp.int32)
    lsb = jnp.bitwise_and(lax.shift_right_logical(t, 16), 1)
    return jnp.bitwise_and(t + 0x7FFF + lsb, _HI)


def _three_nn_body(xyz1_ref, xyz2t_ref, p1_ref, idx_ref, w_ref, tab_ref):
    b = pl.program_id(0)
    pts = p1_ref[0]
    lo = _bf16_topbits(pts[:, :CW])
    hi = _bf16_topbits(pts[:, CW:])
    tab_ref[...] = jnp.bitwise_or(lax.shift_right_logical(lo, 16), hi)

    xyz1 = xyz1_ref[0]
    q = xyz2t_ref[0]
    inf = jnp.float32(jnp.inf)
    RC = 128
    iota_l = lax.broadcasted_iota(jnp.int32, (RC, T2), 0)
    best_v = [jnp.full((1, T2), inf, jnp.float32) for _ in range(3)]
    best_i = [jnp.full((1, T2), N1, jnp.int32) for _ in range(3)]
    for c in range(N1 // RC):
        xc = xyz1[c * RC:(c + 1) * RC]
        dx = xc[:, 0:1] - q[0:1, :]
        dy = xc[:, 1:2] - q[1:2, :]
        dz = xc[:, 2:3] - q[2:3, :]
        cur = (dx * dx + dy * dy) + dz * dz
        for _ in range(3):
            m = jnp.min(cur, axis=0, keepdims=True)
            cand = jnp.where(cur == m, iota_l, RC)
            il = jnp.min(cand, axis=0, keepdims=True)
            cur = jnp.where(iota_l == il, inf, cur)
            ig = il + c * RC
            for j in range(3):
                sw = (m < best_v[j]) | ((m == best_v[j]) & (ig < best_i[j]))
                nv = jnp.where(sw, m, best_v[j])
                ni = jnp.where(sw, ig, best_i[j])
                m = jnp.where(sw, best_v[j], m)
                ig = jnp.where(sw, best_i[j], ig)
                best_v[j], best_i[j] = nv, ni
    rcp = [1.0 / jnp.maximum(v, EPS) for v in best_v]
    norm = rcp[0] + rcp[1] + rcp[2]
    gidx = jnp.concatenate([i + b * N1 for i in best_i], axis=0)
    wval = jnp.concatenate([r / norm for r in rcp], axis=0)
    for h in range(T2 // SUBQ):
        sl = slice(h * SUBQ, (h + 1) * SUBQ)
        idx_ref[0, 0, h] = gidx[:, sl]
        w_ref[0, 0, h] = wval[:, sl]


def _three_nn(xyz1, xyz2t, points1, nb):
    return pl.pallas_call(
        _three_nn_body,
        grid=(nb, NT2),
        in_specs=[
            pl.BlockSpec((1, N1, 3), lambda b, t: (b, 0, 0)),
            pl.BlockSpec((1, 3, T2), lambda b, t: (b, 0, t)),
            pl.BlockSpec((1, RPT, C1), lambda b, t: (b, t, 0)),
        ],
        out_specs=[
            pl.BlockSpec((1, 1, T2 // SUBQ, 3, SUBQ), lambda b, t: (b, t, 0, 0, 0)),
            pl.BlockSpec((1, 1, T2 // SUBQ, 3, SUBQ), lambda b, t: (b, t, 0, 0, 0)),
            pl.BlockSpec((RPT, CW), lambda b, t: (b * NT2 + t, 0)),
        ],
        out_shape=[
            jax.ShapeDtypeStruct((nb, NT2, T2 // SUBQ, 3, SUBQ), jnp.int32),
            jax.ShapeDtypeStruct((nb, NT2, T2 // SUBQ, 3, SUBQ), jnp.float32),
            jax.ShapeDtypeStruct((nb * N1, CW), jnp.int32),
        ],
    )(xyz1, xyz2t, points1)


def _sc_interp(table, idx3, w3, nrow, nstep):
    mesh = plsc.VectorSubcoreMesh(core_axis_name="c", subcore_axis_name="s")
    cp = pltpu.CompilerParams()
    if "needs_layout_passes" in pltpu.CompilerParams.__dataclass_fields__:
        cp = dataclasses.replace(cp, needs_layout_passes=False)

    @functools.partial(
        pl.kernel,
        out_type=jax.ShapeDtypeStruct((nrow, CW), jnp.int32),
        mesh=mesh,
        compiler_params=cp,
        scratch_types=[
            pltpu.VMEM((SUBQ, CW), jnp.int32),
            pltpu.VMEM((SUBQ, CW), jnp.int32),
            pltpu.VMEM((SUBQ, CW), jnp.int32),
        ],
    )
    def run(table_hbm, idx_hbm, w_hbm, out_hbm, r0, r1, r2):
        rows = (r0, r1, r2)

        def body(idx_vm, w_vm, out_vm):
            for k in range(3):
                pltpu.sync_copy(table_hbm.at[idx_vm.at[0, k]], rows[k])

            @pl.loop(0, SUBQ)
            def _(qv):
                z16 = jnp.zeros((LANES,), jnp.int32)
                qi = jnp.full((LANES,), qv, jnp.int32)
                w = []
                for k in range(3):
                    wf = plsc.load_gather(
                        w_vm, [z16, jnp.full((LANES,), k, jnp.int32), qi]
                    )
                    w.append(
                        plsc.pack(wf, wf, format=plsc.PackFormat.INTERLEAVED)
                    )
                for c in range(CW // LANES):
                    cs = pl.ds(c * LANES, LANES)
                    acc = w[0] * plsc.bitcast(rows[0][qv, cs], jnp.bfloat16)
                    acc += w[1] * plsc.bitcast(rows[1][qv, cs], jnp.bfloat16)
                    acc += w[2] * plsc.bitcast(rows[2][qv, cs], jnp.bfloat16)
                    out_vm[qv, cs] = plsc.bitcast(acc, jnp.int32)

        pltpu.emit_pipeline(
            body,
            grid=(nstep,),
            in_specs=[
                pl.BlockSpec((1, 3, SUBQ), lambda i: (i, 0, 0)),
                pl.BlockSpec((1, 3, SUBQ), lambda i: (i, 0, 0)),
            ],
            out_specs=[pl.BlockSpec((SUBQ, CW), lambda i: (i, 0))],
            core_axis_name=("c", "s"),
            dimension_semantics=(pltpu.PARALLEL,),
        )(idx_hbm, w_hbm, out_hbm)

    return run(table, idx3, w3)


def _mlp_body(xp_ref, p_ref, w1a_ref, w1b_ref, b1_ref, w2_ref, b2_ref, o_ref):
    wd = xp_ref[...]
    x_lo = lax.bitcast_convert_type(lax.shift_left(wd, 16), jnp.float32)
    x_hi = lax.bitcast_convert_type(jnp.bitwise_and(wd, _HI), jnp.float32)
    h = jnp.dot(x_lo.astype(jnp.bfloat16), w1a_ref[:CW],
                preferred_element_type=jnp.float32)
    h += jnp.dot(x_hi.astype(jnp.bfloat16), w1a_ref[CW:],
                 preferred_element_type=jnp.float32)
    h += jnp.dot(p_ref[...].astype(jnp.bfloat16), w1b_ref[...],
                 preferred_element_type=jnp.float32)
    h = jnp.maximum(h + b1_ref[...], 0.0)
    o = jnp.dot(h.astype(jnp.bfloat16), w2_ref[...],
                preferred_element_type=jnp.float32)
    o_ref[...] = jnp.maximum(o + b2_ref[...], 0.0)


def _mlp(xp, p, w1a, w1b, b1, w2, b2):
    nrow = xp.shape[0]
    return pl.pallas_call(
        _mlp_body,
        grid=(nrow // TM,),
        in_specs=[
            pl.BlockSpec((TM, CW), lambda i: (i, 0)),
            pl.BlockSpec((TM, C2), lambda i: (i, 0)),
            pl.BlockSpec((C1, F1), lambda i: (0, 0)),
            pl.BlockSpec((C2, F1), lambda i: (0, 0)),
            pl.BlockSpec((1, F1), lambda i: (0, 0)),
            pl.BlockSpec((F1, F2), lambda i: (0, 0)),
            pl.BlockSpec((1, F2), lambda i: (0, 0)),
        ],
        out_specs=pl.BlockSpec((TM, F2), lambda i: (i, 0)),
        out_shape=jax.ShapeDtypeStruct((nrow, F2), jnp.float32),
    )(xp, p, w1a, w1b, b1, w2, b2)


GROUPS = (8, 8)


def kernel(inputs_0, inputs_1, W1, b1, W2, b2):
    xyz1 = inputs_0[:, :, 0:3]
    points1 = inputs_0[:, :, 3:]
    xyz2 = inputs_1[:, :, 0:3]
    points2 = inputs_1[:, :, 3:]
    xyz2t = jnp.transpose(xyz2, (0, 2, 1))

    w1a = W1[:C1].astype(jnp.bfloat16)
    w1b = W1[C1:].astype(jnp.bfloat16)
    w2c = W2.astype(jnp.bfloat16)
    b1r, b2r = b1.reshape(1, F1), b2.reshape(1, F2)

    hs = []
    base = 0
    for bg in GROUPS:
        sl = slice(base, base + bg)
        base += bg
        idx3, w3, table = _three_nn(xyz1[sl], xyz2t[sl], points1[sl], bg)
        nstep = (bg * N2) // SUBQ
        interp = _sc_interp(
            table,
            idx3.reshape(nstep, 3, SUBQ),
            w3.reshape(nstep, 3, SUBQ),
            bg * N2,
            nstep,
        )
        hs.append(
            _mlp(interp, points2[sl].reshape(bg * N2, C2),
                 w1a, w1b, b1r, w2c, b2r)
        )
    h = jnp.concatenate(hs, axis=0)
    return h.reshape(B, N2, F2), xyz2
